# baseline (speedup 1.0000x reference)
import functools

import jax
import jax.numpy as jnp
from jax import lax
from jax.experimental import pallas as pl
from jax.experimental.pallas import tpu as pltpu
from jax.experimental.pallas import tpu_sc as plsc

NUM_TAGS = 8
NUM_PROMPT_TOKENS = 128
HIDDEN = 4096

NC, NS, L = 2, 16, 16
NW = NC * NS
PER_W = NUM_PROMPT_TOKENS // NW
G = 2

_mesh = plsc.VectorSubcoreMesh(core_axis_name="c", subcore_axis_name="s")


@functools.partial(
    pl.kernel,
    mesh=_mesh,
    out_type=jax.ShapeDtypeStruct((NUM_PROMPT_TOKENS, HIDDEN), jnp.float32),
    scratch_types=[
        pltpu.VMEM((2 * L,), jnp.int32),
        pltpu.VMEM((G, HIDDEN), jnp.float32),
        pltpu.VMEM((G, HIDDEN), jnp.float32),
        pltpu.VMEM((G, HIDDEN), jnp.float32),
        pltpu.VMEM((G, HIDDEN), jnp.float32),
        pltpu.SemaphoreType.DMA,
        pltpu.SemaphoreType.DMA,
        pltpu.SemaphoreType.DMA,
        pltpu.SemaphoreType.DMA,
    ],
)
def _prompt_table_sc(pid_hbm, pt_hbm, pos_hbm, out_hbm,
                     idx_v, a0_v, b0_v, a1_v, b1_v, s0, s1, s2, s3):
    wid = lax.axis_index("s") * NC + lax.axis_index("c")
    pltpu.sync_copy(pid_hbm, idx_v.at[pl.ds(0, L)])
    pid_vec = idx_v[pl.ds(0, L)]
    lane2 = jnp.minimum(lax.iota(jnp.int32, L), G - 1)
    base = pid_vec * NUM_PROMPT_TOKENS + wid * PER_W
    idx_v[pl.ds(0, L)] = base + lane2
    idx_v[pl.ds(L, L)] = base + G + lane2

    g0a = pltpu.make_async_copy(pt_hbm.at[idx_v.at[pl.ds(0, G)]], a0_v, s0)
    g0b = pltpu.make_async_copy(pos_hbm.at[idx_v.at[pl.ds(0, G)]], b0_v, s1)
    g1a = pltpu.make_async_copy(pt_hbm.at[idx_v.at[pl.ds(L, G)]], a1_v, s2)
    g1b = pltpu.make_async_copy(pos_hbm.at[idx_v.at[pl.ds(L, G)]], b1_v, s3)
    g0a.start()
    g0b.start()
    g1a.start()
    g1b.start()

    def add_rows(a_v, b_v):
        for r in range(G):
            def add_chunk(i, _, r=r):
                for j in range(16):
                    sl = pl.ds((i * 16 + j) * L, L)
                    plsc.addupdate(a_v.at[r, sl], b_v[r, sl])
                return 0
            lax.fori_loop(0, HIDDEN // L // 16, add_chunk, 0)

    g0a.wait()
    g0b.wait()
    add_rows(a0_v, b0_v)
    wb0 = pltpu.make_async_copy(a0_v, out_hbm.at[pl.ds(wid * PER_W, G)], s0)
    wb0.start()
    g1a.wait()
    g1b.wait()
    add_rows(a1_v, b1_v)
    wb1 = pltpu.make_async_copy(a1_v, out_hbm.at[pl.ds(wid * PER_W + G, G)], s1)
    wb1.start()
    wb0.wait()
    wb1.wait()


def kernel(prompt_id, prompt_tables, position_tables):
    pid16 = jnp.broadcast_to(prompt_id, (L,))
    pt = prompt_tables.reshape(NUM_TAGS * NUM_PROMPT_TOKENS, HIDDEN)
    pos = position_tables.reshape(NUM_TAGS * NUM_PROMPT_TOKENS, HIDDEN)
    return _prompt_table_sc(pid16, pt, pos)


# speedup vs baseline: 1.0121x; 1.0121x over previous
import functools

import jax
import jax.numpy as jnp
from jax import lax
from jax.experimental import pallas as pl
from jax.experimental.pallas import tpu as pltpu
from jax.experimental.pallas import tpu_sc as plsc

NUM_TAGS = 8
NUM_PROMPT_TOKENS = 128
HIDDEN = 4096

NC, NS, L = 2, 16, 16
NW = NC * NS
PER_W = NUM_PROMPT_TOKENS // NW
G = 2

_mesh = plsc.VectorSubcoreMesh(core_axis_name="c", subcore_axis_name="s")


@functools.partial(
    pl.kernel,
    mesh=_mesh,
    out_type=jax.ShapeDtypeStruct((NUM_PROMPT_TOKENS, HIDDEN), jnp.float32),
    scratch_types=[
        pltpu.VMEM((2 * L,), jnp.int32),
        pltpu.VMEM((G, HIDDEN), jnp.float32),
        pltpu.VMEM((G, HIDDEN), jnp.float32),
        pltpu.VMEM((G, HIDDEN), jnp.float32),
        pltpu.VMEM((G, HIDDEN), jnp.float32),
        pltpu.SemaphoreType.DMA,
        pltpu.SemaphoreType.DMA,
        pltpu.SemaphoreType.DMA,
        pltpu.SemaphoreType.DMA,
    ],
)
def _prompt_table_sc(pid_hbm, pt_hbm, pos_hbm, out_hbm,
                     idx_v, a0_v, b0_v, a1_v, b1_v, s0, s1, s2, s3):
    wid = lax.axis_index("s") * NC + lax.axis_index("c")
    pltpu.sync_copy(pid_hbm, idx_v.at[pl.ds(0, L)])
    pid_vec = idx_v[pl.ds(0, L)]
    lane2 = jnp.minimum(lax.iota(jnp.int32, L), G - 1)
    base = pid_vec * NUM_PROMPT_TOKENS + wid * PER_W
    idx_v[pl.ds(0, L)] = base + lane2
    idx_v[pl.ds(L, L)] = base + G + lane2

    g0a = pltpu.make_async_copy(pt_hbm.at[idx_v.at[pl.ds(0, G)]], a0_v, s0)
    g0b = pltpu.make_async_copy(pos_hbm.at[idx_v.at[pl.ds(0, G)]], b0_v, s1)
    g1a = pltpu.make_async_copy(pt_hbm.at[idx_v.at[pl.ds(L, G)]], a1_v, s2)
    g1b = pltpu.make_async_copy(pos_hbm.at[idx_v.at[pl.ds(L, G)]], b1_v, s3)
    g0a.start()
    g0b.start()
    g1a.start()
    g1b.start()

    def add_rows(a_v, b_v):
        for r in range(G):
            @plsc.parallel_loop(0, HIDDEN, step=L, unroll=8)
            def _body(i, r=r):
                plsc.addupdate(a_v.at[r, pl.ds(i, L)], b_v[r, pl.ds(i, L)])

    g0a.wait()
    g0b.wait()
    add_rows(a0_v, b0_v)
    wb0 = pltpu.make_async_copy(a0_v, out_hbm.at[pl.ds(wid * PER_W, G)], s0)
    wb0.start()
    g1a.wait()
    g1b.wait()
    add_rows(a1_v, b1_v)
    wb1 = pltpu.make_async_copy(a1_v, out_hbm.at[pl.ds(wid * PER_W + G, G)], s1)
    wb1.start()
    wb0.wait()
    wb1.wait()


def kernel(prompt_id, prompt_tables, position_tables):
    pid16 = jnp.broadcast_to(prompt_id, (L,))
    pt = prompt_tables.reshape(NUM_TAGS * NUM_PROMPT_TOKENS, HIDDEN)
    pos = position_tables.reshape(NUM_TAGS * NUM_PROMPT_TOKENS, HIDDEN)
    return _prompt_table_sc(pid16, pt, pos)


# speedup vs baseline: 1.0409x; 1.0285x over previous
import functools

import jax
import jax.numpy as jnp
from jax import lax
from jax.experimental import pallas as pl
from jax.experimental.pallas import tpu as pltpu
from jax.experimental.pallas import tpu_sc as plsc

NUM_TAGS = 8
NUM_PROMPT_TOKENS = 128
HIDDEN = 4096

NC, NS, L = 2, 16, 16
NW = NC * NS
PER_W = NUM_PROMPT_TOKENS // NW

_mesh = plsc.VectorSubcoreMesh(core_axis_name="c", subcore_axis_name="s")


@functools.partial(
    pl.kernel,
    mesh=_mesh,
    out_type=jax.ShapeDtypeStruct((NUM_PROMPT_TOKENS, HIDDEN), jnp.float32),
    scratch_types=(
        [pltpu.VMEM((2 * L,), jnp.int32)]
        + [pltpu.VMEM((1, HIDDEN), jnp.float32)] * (2 * PER_W)
        + [pltpu.SemaphoreType.DMA] * (2 * PER_W)
    ),
)
def _prompt_table_sc(pid_hbm, pt_hbm, pos_hbm, out_hbm, idx_v, *rest):
    bufs = rest[:2 * PER_W]
    sems = rest[2 * PER_W:]
    a_bufs, b_bufs = bufs[:PER_W], bufs[PER_W:]
    a_sems, b_sems = sems[:PER_W], sems[PER_W:]

    wid = lax.axis_index("s") * NC + lax.axis_index("c")
    pltpu.sync_copy(pid_hbm, idx_v.at[pl.ds(0, L)])
    pid_vec = idx_v[pl.ds(0, L)]
    base = pid_vec * NUM_PROMPT_TOKENS + wid * PER_W
    half = lax.shift_right_logical(lax.iota(jnp.int32, L), 3)
    idx_v[pl.ds(0, L)] = base + half
    idx_v[pl.ds(L, L)] = base + 2 + half

    gathers = []
    for r in range(PER_W):
        ir = idx_v.at[pl.ds(8 * r, 1)]
        ga = pltpu.make_async_copy(pt_hbm.at[ir], a_bufs[r], a_sems[r])
        gb = pltpu.make_async_copy(pos_hbm.at[ir], b_bufs[r], b_sems[r])
        ga.start()
        gb.start()
        gathers.append((ga, gb))

    wbs = []
    for r in range(PER_W):
        ga, gb = gathers[r]
        ga.wait()
        gb.wait()
        a_v, b_v = a_bufs[r], b_bufs[r]

        @plsc.parallel_loop(0, HIDDEN, step=L, unroll=8)
        def _body(i, a_v=a_v, b_v=b_v):
            plsc.addupdate(a_v.at[0, pl.ds(i, L)], b_v[0, pl.ds(i, L)])

        wb = pltpu.make_async_copy(
            a_v, out_hbm.at[pl.ds(wid * PER_W + r, 1)], a_sems[r])
        wb.start()
        wbs.append(wb)
    for wb in wbs:
        wb.wait()


def kernel(prompt_id, prompt_tables, position_tables):
    pid16 = jnp.broadcast_to(prompt_id, (L,))
    pt = prompt_tables.reshape(NUM_TAGS * NUM_PROMPT_TOKENS, HIDDEN)
    pos = position_tables.reshape(NUM_TAGS * NUM_PROMPT_TOKENS, HIDDEN)
    return _prompt_table_sc(pid16, pt, pos)
